# Initial kernel scaffold; baseline (speedup 1.0000x reference)
import jax
import jax.numpy as jnp
from jax.experimental import pallas as pl


def kernel(node_feat, edge_attr, edge_index, batch, Wn, bn, We, be, spreadlayer, strong, uW1, ub1, uW2, ub2, hW1, hb1, hW2, hb2):
    raise NotImplementedError("write your pallas kernel here")


# speedup vs baseline: 276.9281x; 276.9281x over previous
import functools

import jax
import jax.numpy as jnp
from jax.experimental import pallas as pl
from jax.experimental.pallas import tpu as pltpu

_B = 1024


def _v_body(cps_ref, out_ref):
    f32 = jnp.float32
    ri = jax.lax.broadcasted_iota(jnp.int32, (8, 8), 0)
    ci = jax.lax.broadcasted_iota(jnp.int32, (8, 8), 1)
    Ur = jnp.where(ri == ci, 1.0, 0.0).astype(f32)
    Ui = jnp.zeros((8, 8), f32)

    def xor_roll(X, p):
        m = 1 << p
        is0 = ((ri >> p) & 1) == 0
        return jnp.where(is0, jnp.roll(X, -m, axis=0), jnp.roll(X, m, axis=0))

    def rot_gate(Ur, Ui, g, w):
        c = cps_ref[g * 6 + 0]
        s = cps_ref[g * 6 + 1]
        cp = cps_ref[g * 6 + 2]
        sp = cps_ref[g * 6 + 3]
        cm = cps_ref[g * 6 + 4]
        sm = cps_ref[g * 6 + 5]
        g00 = (c * cp, -c * sp)
        g01 = (-s * cm, s * sm)
        g10 = (s * cm, s * sm)
        g11 = (c * cp, c * sp)
        p = 2 - w
        is0 = ((ri >> p) & 1) == 0
        Sr = xor_roll(Ur, p)
        Si = xor_roll(Ui, p)
        t0r = g00[0] * Ur - g00[1] * Ui + g01[0] * Sr - g01[1] * Si
        t0i = g00[0] * Ui + g00[1] * Ur + g01[0] * Si + g01[1] * Sr
        t1r = g10[0] * Sr - g10[1] * Si + g11[0] * Ur - g11[1] * Ui
        t1i = g10[0] * Si + g10[1] * Sr + g11[0] * Ui + g11[1] * Ur
        return jnp.where(is0, t0r, t1r), jnp.where(is0, t0i, t1i)

    def cnot_gate(Ur, Ui, cw, tw):
        pc = 2 - cw
        pt = 2 - tw
        ctrl = ((ri >> pc) & 1) == 1
        return (jnp.where(ctrl, xor_roll(Ur, pt), Ur),
                jnp.where(ctrl, xor_roll(Ui, pt), Ui))

    for l in range(2):
        for j in range(3):
            Ur, Ui = rot_gate(Ur, Ui, l * 3 + j, j)
        r = (l % 2) + 1
        for j in range(3):
            Ur, Ui = cnot_gate(Ur, Ui, j, (j + r) % 3)
    out_ref[0] = Ur
    out_ref[1] = Ui


def _stage1_body(nfe_ref, e0_ref, e1_ref, e2_ref, wn_ref, we_ref, sc_ref,
                 out_ref):
    f32 = jnp.float32
    dn = (((1,), (1,)), ((), ()))
    nfT = jax.lax.dot_general(wn_ref[...], nfe_ref[...], dn,
                              preferred_element_type=f32)
    efT = [
        jax.lax.dot_general(we_ref[...], e_ref[0], dn,
                            preferred_element_type=f32)
        for e_ref in (e0_ref, e1_ref, e2_ref)
    ]

    def raw4(mat, boff):
        a = [mat[i:i + 1, :] + sc_ref[boff + i] for i in range(3)]
        c = [jnp.cos(0.5 * x) for x in a]
        s = [jnp.sin(0.5 * x) for x in a]
        r0r = c[2] * c[0] * c[1] + s[2] * s[0] * s[1]
        r0i = -(c[2] * c[0] * s[1] + s[2] * s[0] * c[1])
        r1r = c[2] * s[0] * c[1] - s[2] * c[0] * s[1]
        r1i = c[2] * s[0] * s[1] - s[2] * c[0] * c[1]
        return (r0r, r0i, r1r, r1i), a

    def apply_spread(base, r):
        r0r, r0i, r1r, r1i = r
        S = [sc_ref[base + t] for t in range(8)]
        u0r = S[0] * r0r - S[1] * r0i + S[2] * r1r - S[3] * r1i
        u0i = S[0] * r0i + S[1] * r0r + S[2] * r1i + S[3] * r1r
        u1r = S[4] * r0r - S[5] * r0i + S[6] * r1r - S[7] * r1i
        u1i = S[4] * r0i + S[5] * r0r + S[6] * r1i + S[7] * r1r
        return [u0r, u0i, u1r, u1i]

    rows = []
    rn, nfa = raw4(nfT, 0)
    for j in range(4):
        rows += apply_spread(6 + 8 * j, rn)
    for k in range(3):
        rek, _ = raw4(efT[k], 3)
        rows += apply_spread(38 + 8 * k, rek)
    rows += [nfa[0], nfa[1], nfa[2], jnp.zeros_like(nfa[0])]
    out_ref[...] = jnp.concatenate(rows, axis=0)


def _cmul(a, b):
    return (a[0] * b[0] - a[1] * b[1], a[0] * b[1] + a[1] * b[0])


def _cadd(a, b):
    return (a[0] + b[0], a[1] + b[1])


def _conj(a):
    return (a[0], -a[1])


def _stage2_body(big_ref, un1_ref, un2_ref, un3_ref, v_ref, mlp_ref,
                 out_ref, head_ref, *, N, SL, M2):
    pid = pl.program_id(0)

    def state4(ref, r0):
        return ((ref[r0], ref[r0 + 1]), (ref[r0 + 2], ref[r0 + 3]))

    un = [state4(big_ref, 0), state4(un1_ref, 0), state4(un2_ref, 0),
          state4(un3_ref, 0)]
    ue = [state4(big_ref, 16 + 4 * k) for k in range(3)]

    def vc(x, u_, y, a, v, b):
        i = (x * 4 + u_ * 2 + y) * 8 + (a * 4 + v * 2 + b)
        return (v_ref[0, i], v_ref[1, i])

    def block_A(e, n):
        P = [[_cmul(e[a], n[b]) for b in range(2)] for a in range(2)]
        A = {}
        for x in range(2):
            for y in range(2):
                for u_ in range(2):
                    for v in range(2):
                        acc = None
                        for a in range(2):
                            for b in range(2):
                                t = _cmul(vc(x, u_, y, a, v, b), P[a][b])
                                acc = t if acc is None else _cadd(acc, t)
                        A[(x, y, u_, v)] = acc
        return A

    u0 = un[0]
    rho = {(v, z): _cmul(u0[v], _conj(u0[z])) for v in range(2)
           for z in range(2)}

    for m in range(2):
        A = block_A(ue[m], un[m + 1])
        B1 = {}
        for x in range(2):
            for y in range(2):
                for u_ in range(2):
                    for z in range(2):
                        acc = None
                        for v in range(2):
                            t = _cmul(A[(x, y, u_, v)], rho[(v, z)])
                            acc = t if acc is None else _cadd(acc, t)
                        B1[(x, y, u_, z)] = acc
        rho_new = {}
        for u_ in range(2):
            for w in range(2):
                acc = None
                for x in range(2):
                    for y in range(2):
                        for z in range(2):
                            t = _cmul(B1[(x, y, u_, z)],
                                      _conj(A[(x, y, w, z)]))
                            acc = t if acc is None else _cadd(acc, t)
                rho_new[(u_, w)] = acc
        rho = rho_new

    A2 = block_A(ue[2], un[3])
    p = []
    for b in range(2):
        acc = None
        for x in range(2):
            for u_ in range(2):
                for z in range(2):
                    s = None
                    for v in range(2):
                        t = _cmul(A2[(x, b, u_, v)], rho[(v, z)])
                        s = t if s is None else _cadd(s, t)
                    a2c = A2[(x, b, u_, z)]
                    term = s[0] * a2c[0] + s[1] * a2c[1]
                    acc = term if acc is None else acc + term
        p.append(acc)

    h = [big_ref[28], big_ref[29], big_ref[30], p[0], p[1]]
    y = []
    for j in range(3):
        acc = None
        for k in range(5):
            t = mlp_ref[j * 5 + k] * h[k]
            acc = t if acc is None else acc + t
        acc = acc + mlp_ref[15 + j]
        y.append(jnp.where(acc >= 0, acc, 0.01 * acc))
    z_rows = []
    for j in range(3):
        acc = None
        for k in range(3):
            t = mlp_ref[18 + j * 3 + k] * y[k]
            acc = t if acc is None else acc + t
        acc = acc + mlp_ref[27 + j]
        z_rows.append(jnp.maximum(acc, 0.0))

    s_iota = jax.lax.broadcasted_iota(jnp.int32, (8, 128), 0)
    l_iota = jax.lax.broadcasted_iota(jnp.int32, (8, 128), 1)
    node_id = s_iota * SL + l_iota + pid * 128
    valid = node_id < N

    @pl.when(pid == 0)
    def _():
        out_ref[...] = jnp.zeros((4, 8, 128), jnp.float32)

    for j in range(3):
        out_ref[j] = out_ref[j] + jnp.where(valid, z_rows[j], 0.0)

    @pl.when(pid == M2 - 1)
    def _():
        g = [jnp.sum(out_ref[j]) / N for j in range(3)]
        l1 = []
        for i in range(2):
            acc = mlp_ref[30 + i * 3] * g[0] + mlp_ref[31 + i * 3] * g[1] \
                + mlp_ref[32 + i * 3] * g[2] + mlp_ref[36 + i]
            l1.append(jnp.where(acc >= 0, acc, 0.01 * acc))
        o = []
        for k in range(2):
            o.append(mlp_ref[38 + k * 2] * l1[0]
                     + mlp_ref[39 + k * 2] * l1[1] + mlp_ref[42 + k])
        head_ref[...] = jnp.where(
            (s_iota == 0) & (l_iota == 0), o[0],
            jnp.where((s_iota == 0) & (l_iota == 1), o[1], 0.0))


def kernel(node_feat, edge_attr, edge_index, batch, Wn, bn, We, be,
           spreadlayer, strong, uW1, ub1, uW2, ub2, hW1, hb1, hW2, hb2):
    f32 = jnp.float32
    N = node_feat.shape[0]
    B = _B
    M2 = -(-N // B)
    NP = M2 * B
    M1 = M2 + 1
    NV = M1 * B
    pad = NV - N
    SL = NP // 8
    E = edge_attr.shape[1]
    F = Wn.shape[1]

    nfe = jnp.concatenate([node_feat, node_feat[:pad]], axis=0)
    eaT = edge_attr.reshape(N, 3, E).transpose(1, 0, 2)
    ea3 = jnp.concatenate([eaT, eaT[:, :pad]], axis=1)

    Wn8 = jnp.pad(Wn, ((0, 5), (0, 0)))
    We8 = jnp.pad(We, ((0, 5), (0, 0)))

    s7 = jnp.concatenate([spreadlayer[1, 3:7], spreadlayer[0, 0:3]])
    ch, sh = jnp.cos(s7 / 2), jnp.sin(s7 / 2)
    ent = jnp.stack([ch * ch, -ch * sh, -sh * ch, sh * sh,
                     sh * ch, sh * sh, ch * ch, ch * sh], axis=1)
    sc = jnp.concatenate([bn, be, ent.reshape(56)]).astype(f32)

    W = strong[0]
    th, phi, om = W[..., 1], W[..., 0], W[..., 2]
    ap, am = (om + phi) / 2, (om - phi) / 2
    cps = jnp.stack([jnp.cos(th / 2), jnp.sin(th / 2),
                     jnp.cos(ap), jnp.sin(ap),
                     jnp.cos(am), jnp.sin(am)], axis=-1).reshape(36)

    vout = pl.pallas_call(
        _v_body,
        grid=(1,),
        in_specs=[pl.BlockSpec(memory_space=pltpu.SMEM)],
        out_specs=pl.BlockSpec((2, 8, 8), lambda i: (0, 0, 0)),
        out_shape=jax.ShapeDtypeStruct((2, 8, 8), f32),
    )(cps.astype(f32))
    v_flat = vout.reshape(2, 64)

    out1 = pl.pallas_call(
        _stage1_body,
        grid=(M1,),
        in_specs=[
            pl.BlockSpec((B, F), lambda i: (i, 0)),
            pl.BlockSpec((1, B, E), lambda i: (0, i, 0)),
            pl.BlockSpec((1, B, E), lambda i: (1, i, 0)),
            pl.BlockSpec((1, B, E), lambda i: (2, i, 0)),
            pl.BlockSpec((8, F), lambda i: (0, 0)),
            pl.BlockSpec((8, E), lambda i: (0, 0)),
            pl.BlockSpec(memory_space=pltpu.SMEM),
        ],
        out_specs=pl.BlockSpec((32, B), lambda i: (0, i)),
        out_shape=jax.ShapeDtypeStruct((32, NV), f32),
    )(nfe, ea3, ea3, ea3, Wn8, We8, sc)

    big = jax.lax.slice(out1, (0, 0), (32, NP)).reshape(32, 8, SL)

    def strip(r0, j):
        v = jax.lax.slice(out1, (r0, j), (r0 + 4, j + NP))
        return v.reshape(4, 8, SL)

    un_views = [strip(4 * j, j) for j in range(1, 4)]

    mlp_s = jnp.concatenate(
        [uW1.reshape(-1), ub1, uW2.reshape(-1), ub2,
         hW1.reshape(-1), hb1, hW2.reshape(-1), hb2]).astype(f32)

    _, head = pl.pallas_call(
        functools.partial(_stage2_body, N=N, SL=SL, M2=M2),
        grid=(M2,),
        in_specs=[pl.BlockSpec((32, 8, 128), lambda i: (0, 0, i))]
        + [pl.BlockSpec((4, 8, 128), lambda i: (0, 0, i))] * 3
        + [pl.BlockSpec(memory_space=pltpu.SMEM)] * 2,
        out_specs=[pl.BlockSpec((4, 8, 128), lambda i: (0, 0, 0)),
                   pl.BlockSpec((8, 128), lambda i: (0, 0))],
        out_shape=[jax.ShapeDtypeStruct((4, 8, 128), f32),
                   jax.ShapeDtypeStruct((8, 128), f32)],
    )(big, *un_views, v_flat, mlp_s)

    return head[0:1, 0:2]


# speedup vs baseline: 327.4240x; 1.1823x over previous
import functools

import jax
import jax.numpy as jnp
from jax.experimental import pallas as pl
from jax.experimental.pallas import tpu as pltpu

_B = 1024


def _v_build(cps_ref):
    f32 = jnp.float32
    ri = jax.lax.broadcasted_iota(jnp.int32, (8, 8), 0)
    ci = jax.lax.broadcasted_iota(jnp.int32, (8, 8), 1)
    Ur = jnp.where(ri == ci, 1.0, 0.0).astype(f32)
    Ui = jnp.zeros((8, 8), f32)

    def xor_roll(X, p):
        m = 1 << p
        is0 = ((ri >> p) & 1) == 0
        return jnp.where(is0, jnp.roll(X, -m, axis=0), jnp.roll(X, m, axis=0))

    def rot_gate(Ur, Ui, g, w):
        c = cps_ref[g * 6 + 0]
        s = cps_ref[g * 6 + 1]
        cp = cps_ref[g * 6 + 2]
        sp = cps_ref[g * 6 + 3]
        cm = cps_ref[g * 6 + 4]
        sm = cps_ref[g * 6 + 5]
        g00 = (c * cp, -c * sp)
        g01 = (-s * cm, s * sm)
        g10 = (s * cm, s * sm)
        g11 = (c * cp, c * sp)
        p = 2 - w
        is0 = ((ri >> p) & 1) == 0
        Sr = xor_roll(Ur, p)
        Si = xor_roll(Ui, p)
        t0r = g00[0] * Ur - g00[1] * Ui + g01[0] * Sr - g01[1] * Si
        t0i = g00[0] * Ui + g00[1] * Ur + g01[0] * Si + g01[1] * Sr
        t1r = g10[0] * Sr - g10[1] * Si + g11[0] * Ur - g11[1] * Ui
        t1i = g10[0] * Si + g10[1] * Sr + g11[0] * Ui + g11[1] * Ur
        return jnp.where(is0, t0r, t1r), jnp.where(is0, t0i, t1i)

    def cnot_gate(Ur, Ui, cw, tw):
        pc = 2 - cw
        pt = 2 - tw
        ctrl = ((ri >> pc) & 1) == 1
        return (jnp.where(ctrl, xor_roll(Ur, pt), Ur),
                jnp.where(ctrl, xor_roll(Ui, pt), Ui))

    for l in range(2):
        for j in range(3):
            Ur, Ui = rot_gate(Ur, Ui, l * 3 + j, j)
        r = (l % 2) + 1
        for j in range(3):
            Ur, Ui = cnot_gate(Ur, Ui, j, (j + r) % 3)
    return Ur, Ui


def _stage1_body(nfe_ref, e0_ref, e1_ref, e2_ref, wn_ref, we_ref, sc_ref,
                 cps_ref, out_ref, vout_ref):
    f32 = jnp.float32
    dn = (((1,), (1,)), ((), ()))
    nfT = jax.lax.dot_general(wn_ref[...], nfe_ref[...], dn,
                              preferred_element_type=f32)
    efT = [
        jax.lax.dot_general(we_ref[...], e_ref[0], dn,
                            preferred_element_type=f32)
        for e_ref in (e0_ref, e1_ref, e2_ref)
    ]

    ang = jnp.concatenate([nfT[0:3], efT[0][0:3], efT[1][0:3], efT[2][0:3]],
                          axis=0)
    bias = jnp.stack([sc_ref[0], sc_ref[1], sc_ref[2]]
                     + [sc_ref[3], sc_ref[4], sc_ref[5]] * 3)[:, None]
    half = (ang + bias) * 0.5
    C = jnp.cos(half)
    S = jnp.sin(half)

    def raw4(gi):
        c = [C[3 * gi + i: 3 * gi + i + 1, :] for i in range(3)]
        s = [S[3 * gi + i: 3 * gi + i + 1, :] for i in range(3)]
        r0r = c[2] * c[0] * c[1] + s[2] * s[0] * s[1]
        r0i = -(c[2] * c[0] * s[1] + s[2] * s[0] * c[1])
        r1r = c[2] * s[0] * c[1] - s[2] * c[0] * s[1]
        r1i = c[2] * s[0] * s[1] - s[2] * c[0] * c[1]
        return (r0r, r0i, r1r, r1i)

    def apply_spread(base, r):
        r0r, r0i, r1r, r1i = r
        S = [sc_ref[base + t] for t in range(8)]
        u0r = S[0] * r0r - S[1] * r0i + S[2] * r1r - S[3] * r1i
        u0i = S[0] * r0i + S[1] * r0r + S[2] * r1i + S[3] * r1r
        u1r = S[4] * r0r - S[5] * r0i + S[6] * r1r - S[7] * r1i
        u1i = S[4] * r0i + S[5] * r0r + S[6] * r1i + S[7] * r1r
        return [u0r, u0i, u1r, u1i]

    rows = []
    rn = raw4(0)
    for j in range(4):
        rows += apply_spread(6 + 8 * j, rn)
    for k in range(3):
        rek = raw4(1 + k)
        rows += apply_spread(38 + 8 * k, rek)
    nfa = [nfT[i:i + 1, :] + sc_ref[i] for i in range(3)]
    rows += [nfa[0], nfa[1], nfa[2], jnp.zeros_like(nfa[0])]
    out_ref[...] = jnp.concatenate(rows, axis=0)

    @pl.when(pl.program_id(0) == 0)
    def _():
        Ur, Ui = _v_build(cps_ref)
        vout_ref[0] = Ur
        vout_ref[1] = Ui


def _cmul(a, b):
    return (a[0] * b[0] - a[1] * b[1], a[0] * b[1] + a[1] * b[0])


def _cadd(a, b):
    return (a[0] + b[0], a[1] + b[1])


def _conj(a):
    return (a[0], -a[1])


def _stage2_body(big_ref, un1_ref, un2_ref, un3_ref, v_ref, mlp_ref,
                 out_ref, head_ref, *, N, SL, M2):
    pid = pl.program_id(0)

    def state4(ref, r0):
        return ((ref[r0], ref[r0 + 1]), (ref[r0 + 2], ref[r0 + 3]))

    un = [state4(big_ref, 0), state4(un1_ref, 0), state4(un2_ref, 0),
          state4(un3_ref, 0)]
    ue = [state4(big_ref, 16 + 4 * k) for k in range(3)]

    def vc(x, u_, y, a, v, b):
        i = (x * 4 + u_ * 2 + y) * 8 + (a * 4 + v * 2 + b)
        return (v_ref[0, i], v_ref[1, i])

    def block_A(e, n):
        P = [[_cmul(e[a], n[b]) for b in range(2)] for a in range(2)]
        A = {}
        for x in range(2):
            for y in range(2):
                for u_ in range(2):
                    for v in range(2):
                        acc = None
                        for a in range(2):
                            for b in range(2):
                                t = _cmul(vc(x, u_, y, a, v, b), P[a][b])
                                acc = t if acc is None else _cadd(acc, t)
                        A[(x, y, u_, v)] = acc
        return A

    u0 = un[0]
    rho = {(v, z): _cmul(u0[v], _conj(u0[z])) for v in range(2)
           for z in range(2)}

    for m in range(2):
        A = block_A(ue[m], un[m + 1])
        B1 = {}
        for x in range(2):
            for y in range(2):
                for u_ in range(2):
                    for z in range(2):
                        acc = None
                        for v in range(2):
                            t = _cmul(A[(x, y, u_, v)], rho[(v, z)])
                            acc = t if acc is None else _cadd(acc, t)
                        B1[(x, y, u_, z)] = acc
        rho_new = {}
        for u_ in range(2):
            for w in range(2):
                acc = None
                for x in range(2):
                    for y in range(2):
                        for z in range(2):
                            t = _cmul(B1[(x, y, u_, z)],
                                      _conj(A[(x, y, w, z)]))
                            acc = t if acc is None else _cadd(acc, t)
                rho_new[(u_, w)] = acc
        rho = rho_new

    A2 = block_A(ue[2], un[3])
    p = []
    for b in range(2):
        acc = None
        for x in range(2):
            for u_ in range(2):
                for z in range(2):
                    s = None
                    for v in range(2):
                        t = _cmul(A2[(x, b, u_, v)], rho[(v, z)])
                        s = t if s is None else _cadd(s, t)
                    a2c = A2[(x, b, u_, z)]
                    term = s[0] * a2c[0] + s[1] * a2c[1]
                    acc = term if acc is None else acc + term
        p.append(acc)

    h = [big_ref[28], big_ref[29], big_ref[30], p[0], p[1]]
    y = []
    for j in range(3):
        acc = None
        for k in range(5):
            t = mlp_ref[j * 5 + k] * h[k]
            acc = t if acc is None else acc + t
        acc = acc + mlp_ref[15 + j]
        y.append(jnp.where(acc >= 0, acc, 0.01 * acc))
    z_rows = []
    for j in range(3):
        acc = None
        for k in range(3):
            t = mlp_ref[18 + j * 3 + k] * y[k]
            acc = t if acc is None else acc + t
        acc = acc + mlp_ref[27 + j]
        z_rows.append(jnp.maximum(acc, 0.0))

    s_iota = jax.lax.broadcasted_iota(jnp.int32, (8, 128), 0)
    l_iota = jax.lax.broadcasted_iota(jnp.int32, (8, 128), 1)
    node_id = s_iota * SL + l_iota + pid * 128
    valid = node_id < N

    @pl.when(pid == 0)
    def _():
        out_ref[...] = jnp.zeros((4, 8, 128), jnp.float32)

    for j in range(3):
        out_ref[j] = out_ref[j] + jnp.where(valid, z_rows[j], 0.0)

    @pl.when(pid == M2 - 1)
    def _():
        g = [jnp.sum(out_ref[j]) / N for j in range(3)]
        l1 = []
        for i in range(2):
            acc = mlp_ref[30 + i * 3] * g[0] + mlp_ref[31 + i * 3] * g[1] \
                + mlp_ref[32 + i * 3] * g[2] + mlp_ref[36 + i]
            l1.append(jnp.where(acc >= 0, acc, 0.01 * acc))
        o = []
        for k in range(2):
            o.append(mlp_ref[38 + k * 2] * l1[0]
                     + mlp_ref[39 + k * 2] * l1[1] + mlp_ref[42 + k])
        head_ref[...] = jnp.where(
            (s_iota == 0) & (l_iota == 0), o[0],
            jnp.where((s_iota == 0) & (l_iota == 1), o[1], 0.0))


def kernel(node_feat, edge_attr, edge_index, batch, Wn, bn, We, be,
           spreadlayer, strong, uW1, ub1, uW2, ub2, hW1, hb1, hW2, hb2):
    f32 = jnp.float32
    N = node_feat.shape[0]
    B = _B
    M2 = -(-N // B)
    NP = M2 * B
    SL = NP // 8
    E = edge_attr.shape[1]
    F = Wn.shape[1]

    ea3 = edge_attr.reshape(N, 3, E).transpose(1, 0, 2)

    Wn8 = jnp.pad(Wn, ((0, 5), (0, 0)))
    We8 = jnp.pad(We, ((0, 5), (0, 0)))

    s7 = jnp.concatenate([spreadlayer[1, 3:7], spreadlayer[0, 0:3]])
    ch, sh = jnp.cos(s7 / 2), jnp.sin(s7 / 2)
    ent = jnp.stack([ch * ch, -ch * sh, -sh * ch, sh * sh,
                     sh * ch, sh * sh, ch * ch, ch * sh], axis=1)
    sc = jnp.concatenate([bn, be, ent.reshape(56)]).astype(f32)

    W = strong[0]
    th, phi, om = W[..., 1], W[..., 0], W[..., 2]
    ap, am = (om + phi) / 2, (om - phi) / 2
    cps = jnp.stack([jnp.cos(th / 2), jnp.sin(th / 2),
                     jnp.cos(ap), jnp.sin(ap),
                     jnp.cos(am), jnp.sin(am)], axis=-1).reshape(36)

    out1, vout = pl.pallas_call(
        _stage1_body,
        grid=(M2,),
        in_specs=[
            pl.BlockSpec((B, F), lambda i: (i, 0)),
            pl.BlockSpec((1, B, E), lambda i: (0, i, 0)),
            pl.BlockSpec((1, B, E), lambda i: (1, i, 0)),
            pl.BlockSpec((1, B, E), lambda i: (2, i, 0)),
            pl.BlockSpec((8, F), lambda i: (0, 0)),
            pl.BlockSpec((8, E), lambda i: (0, 0)),
            pl.BlockSpec(memory_space=pltpu.SMEM),
            pl.BlockSpec(memory_space=pltpu.SMEM),
        ],
        out_specs=[pl.BlockSpec((32, B), lambda i: (0, i)),
                   pl.BlockSpec((2, 8, 8), lambda i: (0, 0, 0))],
        out_shape=[jax.ShapeDtypeStruct((32, NP), f32),
                   jax.ShapeDtypeStruct((2, 8, 8), f32)],
    )(node_feat, ea3, ea3, ea3, Wn8, We8, sc, cps.astype(f32))
    v_flat = vout.reshape(2, 64)

    big = out1.reshape(32, 8, SL)

    def strip(r0, j):
        a = jax.lax.slice(out1, (r0, j), (r0 + 4, N))
        b = jax.lax.slice(out1, (r0, 0), (r0 + 4, j + NP - N))
        return jnp.concatenate([a, b], axis=1).reshape(4, 8, SL)

    un_views = [strip(4 * j, j) for j in range(1, 4)]

    mlp_s = jnp.concatenate(
        [uW1.reshape(-1), ub1, uW2.reshape(-1), ub2,
         hW1.reshape(-1), hb1, hW2.reshape(-1), hb2]).astype(f32)

    _, head = pl.pallas_call(
        functools.partial(_stage2_body, N=N, SL=SL, M2=M2),
        grid=(M2,),
        in_specs=[pl.BlockSpec((32, 8, 128), lambda i: (0, 0, i))]
        + [pl.BlockSpec((4, 8, 128), lambda i: (0, 0, i))] * 3
        + [pl.BlockSpec(memory_space=pltpu.SMEM)] * 2,
        out_specs=[pl.BlockSpec((4, 8, 128), lambda i: (0, 0, 0)),
                   pl.BlockSpec((8, 128), lambda i: (0, 0))],
        out_shape=[jax.ShapeDtypeStruct((4, 8, 128), f32),
                   jax.ShapeDtypeStruct((8, 128), f32)],
    )(big, *un_views, v_flat, mlp_s)

    return head[0:1, 0:2]


# speedup vs baseline: 373.8578x; 1.1418x over previous
import functools

import jax
import jax.numpy as jnp
from jax.experimental import pallas as pl
from jax.experimental.pallas import tpu as pltpu

_B = 1024


def _v_build(cps_ref, base):
    f32 = jnp.float32
    ri = jax.lax.broadcasted_iota(jnp.int32, (8, 8), 0)
    ci = jax.lax.broadcasted_iota(jnp.int32, (8, 8), 1)
    Ur = jnp.where(ri == ci, 1.0, 0.0).astype(f32)
    Ui = jnp.zeros((8, 8), f32)

    def xor_roll(X, p):
        m = 1 << p
        is0 = ((ri >> p) & 1) == 0
        return jnp.where(is0, jnp.roll(X, -m, axis=0), jnp.roll(X, m, axis=0))

    def rot_gate(Ur, Ui, g, w):
        c = cps_ref[base + g * 6 + 0]
        s = cps_ref[base + g * 6 + 1]
        cp = cps_ref[base + g * 6 + 2]
        sp = cps_ref[base + g * 6 + 3]
        cm = cps_ref[base + g * 6 + 4]
        sm = cps_ref[base + g * 6 + 5]
        g00 = (c * cp, -c * sp)
        g01 = (-s * cm, s * sm)
        g10 = (s * cm, s * sm)
        g11 = (c * cp, c * sp)
        p = 2 - w
        is0 = ((ri >> p) & 1) == 0
        Sr = xor_roll(Ur, p)
        Si = xor_roll(Ui, p)
        t0r = g00[0] * Ur - g00[1] * Ui + g01[0] * Sr - g01[1] * Si
        t0i = g00[0] * Ui + g00[1] * Ur + g01[0] * Si + g01[1] * Sr
        t1r = g10[0] * Sr - g10[1] * Si + g11[0] * Ur - g11[1] * Ui
        t1i = g10[0] * Si + g10[1] * Sr + g11[0] * Ui + g11[1] * Ur
        return jnp.where(is0, t0r, t1r), jnp.where(is0, t0i, t1i)

    def cnot_gate(Ur, Ui, cw, tw):
        pc = 2 - cw
        pt = 2 - tw
        ctrl = ((ri >> pc) & 1) == 1
        return (jnp.where(ctrl, xor_roll(Ur, pt), Ur),
                jnp.where(ctrl, xor_roll(Ui, pt), Ui))

    for l in range(2):
        for j in range(3):
            Ur, Ui = rot_gate(Ur, Ui, l * 3 + j, j)
        r = (l % 2) + 1
        for j in range(3):
            Ur, Ui = cnot_gate(Ur, Ui, j, (j + r) % 3)
    return Ur, Ui


def _stage1_body(nfe_ref, ea_ref, wn_ref, we_ref, sc_ref,
                 out_ref, vout_ref):
    f32 = jnp.float32
    dn = (((1,), (1,)), ((), ()))
    nfT = jax.lax.dot_general(wn_ref[...], nfe_ref[...], dn,
                              preferred_element_type=f32)
    efT = [
        jax.lax.dot_general(we_ref[...], ea_ref[:, k, :], dn,
                            preferred_element_type=f32)
        for k in range(3)
    ]

    ang = jnp.concatenate([nfT[0:3], efT[0][0:3], efT[1][0:3], efT[2][0:3]],
                          axis=0)
    bias = jnp.stack([sc_ref[0], sc_ref[1], sc_ref[2]]
                     + [sc_ref[3], sc_ref[4], sc_ref[5]] * 3)[:, None]
    half = (ang + bias) * 0.5
    C = jnp.cos(half)
    S = jnp.sin(half)

    def raw4(gi):
        c = [C[3 * gi + i: 3 * gi + i + 1, :] for i in range(3)]
        s = [S[3 * gi + i: 3 * gi + i + 1, :] for i in range(3)]
        r0r = c[2] * c[0] * c[1] + s[2] * s[0] * s[1]
        r0i = -(c[2] * c[0] * s[1] + s[2] * s[0] * c[1])
        r1r = c[2] * s[0] * c[1] - s[2] * c[0] * s[1]
        r1i = c[2] * s[0] * s[1] - s[2] * c[0] * c[1]
        return (r0r, r0i, r1r, r1i)

    def apply_spread(base, r):
        r0r, r0i, r1r, r1i = r
        S = [sc_ref[base + t] for t in range(8)]
        u0r = S[0] * r0r - S[1] * r0i + S[2] * r1r - S[3] * r1i
        u0i = S[0] * r0i + S[1] * r0r + S[2] * r1i + S[3] * r1r
        u1r = S[4] * r0r - S[5] * r0i + S[6] * r1r - S[7] * r1i
        u1i = S[4] * r0i + S[5] * r0r + S[6] * r1i + S[7] * r1r
        return [u0r, u0i, u1r, u1i]

    rows = []
    rn = raw4(0)
    for j in range(4):
        rows += apply_spread(6 + 8 * j, rn)
    for k in range(3):
        rek = raw4(1 + k)
        rows += apply_spread(38 + 8 * k, rek)
    nfa = [nfT[i:i + 1, :] + sc_ref[i] for i in range(3)]
    rows += [nfa[0], nfa[1], nfa[2], jnp.zeros_like(nfa[0])]
    out_ref[...] = jnp.concatenate(rows, axis=0)

    @pl.when(pl.program_id(0) == 0)
    def _():
        Ur, Ui = _v_build(sc_ref, 62)
        vout_ref[0] = Ur
        vout_ref[1] = Ui


def _cmul(a, b):
    return (a[0] * b[0] - a[1] * b[1], a[0] * b[1] + a[1] * b[0])


def _cadd(a, b):
    return (a[0] + b[0], a[1] + b[1])


def _conj(a):
    return (a[0], -a[1])


def _stage2_body(big_ref, un1_ref, un2_ref, un3_ref, v_ref, mlp_ref,
                 out_ref, head_ref, *, N, SL, M2):
    pid = pl.program_id(0)

    def state4(ref, r0):
        return ((ref[r0], ref[r0 + 1]), (ref[r0 + 2], ref[r0 + 3]))

    un = [state4(big_ref, 0), state4(un1_ref, 0), state4(un2_ref, 0),
          state4(un3_ref, 0)]
    ue = [state4(big_ref, 16 + 4 * k) for k in range(3)]

    def vc(x, u_, y, a, v, b):
        r = x * 4 + u_ * 2 + y
        c = a * 4 + v * 2 + b
        return (v_ref[0, r, c], v_ref[1, r, c])

    def block_A(e, n):
        P = [[_cmul(e[a], n[b]) for b in range(2)] for a in range(2)]
        A = {}
        for x in range(2):
            for y in range(2):
                for u_ in range(2):
                    for v in range(2):
                        acc = None
                        for a in range(2):
                            for b in range(2):
                                t = _cmul(vc(x, u_, y, a, v, b), P[a][b])
                                acc = t if acc is None else _cadd(acc, t)
                        A[(x, y, u_, v)] = acc
        return A

    u0 = un[0]
    rho = {(v, z): _cmul(u0[v], _conj(u0[z])) for v in range(2)
           for z in range(2)}

    for m in range(2):
        A = block_A(ue[m], un[m + 1])
        B1 = {}
        for x in range(2):
            for y in range(2):
                for u_ in range(2):
                    for z in range(2):
                        acc = None
                        for v in range(2):
                            t = _cmul(A[(x, y, u_, v)], rho[(v, z)])
                            acc = t if acc is None else _cadd(acc, t)
                        B1[(x, y, u_, z)] = acc
        rho_new = {}
        for u_ in range(2):
            for w in range(2):
                acc = None
                for x in range(2):
                    for y in range(2):
                        for z in range(2):
                            t = _cmul(B1[(x, y, u_, z)],
                                      _conj(A[(x, y, w, z)]))
                            acc = t if acc is None else _cadd(acc, t)
                rho_new[(u_, w)] = acc
        rho = rho_new

    A2 = block_A(ue[2], un[3])
    p = []
    for b in range(2):
        acc = None
        for x in range(2):
            for u_ in range(2):
                for z in range(2):
                    s = None
                    for v in range(2):
                        t = _cmul(A2[(x, b, u_, v)], rho[(v, z)])
                        s = t if s is None else _cadd(s, t)
                    a2c = A2[(x, b, u_, z)]
                    term = s[0] * a2c[0] + s[1] * a2c[1]
                    acc = term if acc is None else acc + term
        p.append(acc)

    h = [big_ref[28], big_ref[29], big_ref[30], p[0], p[1]]
    y = []
    for j in range(3):
        acc = None
        for k in range(5):
            t = mlp_ref[j * 5 + k] * h[k]
            acc = t if acc is None else acc + t
        acc = acc + mlp_ref[15 + j]
        y.append(jnp.where(acc >= 0, acc, 0.01 * acc))
    z_rows = []
    for j in range(3):
        acc = None
        for k in range(3):
            t = mlp_ref[18 + j * 3 + k] * y[k]
            acc = t if acc is None else acc + t
        acc = acc + mlp_ref[27 + j]
        z_rows.append(jnp.maximum(acc, 0.0))

    s_iota = jax.lax.broadcasted_iota(jnp.int32, (8, 128), 0)
    l_iota = jax.lax.broadcasted_iota(jnp.int32, (8, 128), 1)
    node_id = s_iota * SL + l_iota + pid * 128
    valid = node_id < N

    @pl.when(pid == 0)
    def _():
        out_ref[...] = jnp.zeros((4, 8, 128), jnp.float32)

    for j in range(3):
        out_ref[j] = out_ref[j] + jnp.where(valid, z_rows[j], 0.0)

    @pl.when(pid == M2 - 1)
    def _():
        g = [jnp.sum(out_ref[j]) / N for j in range(3)]
        l1 = []
        for i in range(2):
            acc = mlp_ref[30 + i * 3] * g[0] + mlp_ref[31 + i * 3] * g[1] \
                + mlp_ref[32 + i * 3] * g[2] + mlp_ref[36 + i]
            l1.append(jnp.where(acc >= 0, acc, 0.01 * acc))
        o = []
        for k in range(2):
            o.append(mlp_ref[38 + k * 2] * l1[0]
                     + mlp_ref[39 + k * 2] * l1[1] + mlp_ref[42 + k])
        head_ref[...] = jnp.where(
            (s_iota == 0) & (l_iota == 0), o[0],
            jnp.where((s_iota == 0) & (l_iota == 1), o[1], 0.0))


def kernel(node_feat, edge_attr, edge_index, batch, Wn, bn, We, be,
           spreadlayer, strong, uW1, ub1, uW2, ub2, hW1, hb1, hW2, hb2):
    f32 = jnp.float32
    N = node_feat.shape[0]
    B = _B
    M2 = -(-N // B)
    NP = M2 * B
    SL = NP // 8
    E = edge_attr.shape[1]
    F = Wn.shape[1]

    ea3 = edge_attr.reshape(N, 3, E)

    Wn8 = jnp.pad(Wn, ((0, 5), (0, 0)))
    We8 = jnp.pad(We, ((0, 5), (0, 0)))

    s7 = jnp.concatenate([spreadlayer[1, 3:7], spreadlayer[0, 0:3]])
    ch, sh = jnp.cos(s7 / 2), jnp.sin(s7 / 2)
    ent = jnp.stack([ch * ch, -ch * sh, -sh * ch, sh * sh,
                     sh * ch, sh * sh, ch * ch, ch * sh], axis=1)

    W = strong[0]
    th, phi, om = W[..., 1], W[..., 0], W[..., 2]
    ap, am = (om + phi) / 2, (om - phi) / 2
    cps = jnp.stack([jnp.cos(th / 2), jnp.sin(th / 2),
                     jnp.cos(ap), jnp.sin(ap),
                     jnp.cos(am), jnp.sin(am)], axis=-1).reshape(36)
    sc = jnp.concatenate([bn, be, ent.reshape(56), cps]).astype(f32)

    out1, vout = pl.pallas_call(
        _stage1_body,
        grid=(M2,),
        in_specs=[
            pl.BlockSpec((B, F), lambda i: (i, 0)),
            pl.BlockSpec((B, 3, E), lambda i: (i, 0, 0)),
            pl.BlockSpec((8, F), lambda i: (0, 0)),
            pl.BlockSpec((8, E), lambda i: (0, 0)),
            pl.BlockSpec(memory_space=pltpu.SMEM),
        ],
        out_specs=[pl.BlockSpec((32, B), lambda i: (0, i)),
                   pl.BlockSpec((2, 8, 8), lambda i: (0, 0, 0))],
        out_shape=[jax.ShapeDtypeStruct((32, NP), f32),
                   jax.ShapeDtypeStruct((2, 8, 8), f32)],
    )(node_feat, ea3, Wn8, We8, sc)

    big = out1.reshape(32, 8, SL)

    def strip(r0, j):
        a = jax.lax.slice(out1, (r0, j), (r0 + 4, N))
        b = jax.lax.slice(out1, (r0, 0), (r0 + 4, j + NP - N))
        return jnp.concatenate([a, b], axis=1).reshape(4, 8, SL)

    un_views = [strip(4 * j, j) for j in range(1, 4)]

    mlp_s = jnp.concatenate(
        [uW1.reshape(-1), ub1, uW2.reshape(-1), ub2,
         hW1.reshape(-1), hb1, hW2.reshape(-1), hb2]).astype(f32)

    _, head = pl.pallas_call(
        functools.partial(_stage2_body, N=N, SL=SL, M2=M2),
        grid=(M2,),
        in_specs=[pl.BlockSpec((32, 8, 128), lambda i: (0, 0, i))]
        + [pl.BlockSpec((4, 8, 128), lambda i: (0, 0, i))] * 3
        + [pl.BlockSpec(memory_space=pltpu.SMEM)] * 2,
        out_specs=[pl.BlockSpec((4, 8, 128), lambda i: (0, 0, 0)),
                   pl.BlockSpec((8, 128), lambda i: (0, 0))],
        out_shape=[jax.ShapeDtypeStruct((4, 8, 128), f32),
                   jax.ShapeDtypeStruct((8, 128), f32)],
    )(big, *un_views, vout, mlp_s)

    return head[0:1, 0:2]
